# baseline (speedup 1.0000x reference)
import functools

import jax
import jax.numpy as jnp
from jax import lax
from jax.experimental import pallas as pl
from jax.experimental.pallas import tpu as pltpu
from jax.experimental.pallas import tpu_sc as plsc

BATCH = 4096
HIST_LEN = 200
EMBED_DIM = 32

NUM_CORES = 2
NUM_SUBCORES = 16
NUM_WORKERS = NUM_CORES * NUM_SUBCORES

CHUNK = 512
CB = CHUNK // 128
CPT = BATCH // CHUNK
NUM_UNITS = HIST_LEN * CPT
UPW = NUM_UNITS // NUM_WORKERS
TBLK = 4 * CB * 8 * 128


@functools.partial(
    pl.kernel,
    mesh=plsc.VectorSubcoreMesh(core_axis_name="c", subcore_axis_name="s"),
    out_type=jax.ShapeDtypeStruct((HIST_LEN, 4, 32 * 8 * 128), jnp.float32),
    scratch_types=[
        pltpu.VMEM((2, CHUNK), jnp.int32),
        pltpu.VMEM((2, CHUNK, EMBED_DIM), jnp.float32),
        pltpu.VMEM((2, TBLK), jnp.float32),
        pltpu.SemaphoreType.DMA((2,)),
        pltpu.SemaphoreType.DMA((2,)),
    ],
    compiler_params=pltpu.CompilerParams(use_tc_tiling_on_sc=False,
                                         needs_layout_passes=False),
)
def _embed(x_hbm, table_hbm, out_hbm, idx_v, rows_v, trans_v, gsem, ssem):
    wid = lax.axis_index("s") * NUM_CORES + lax.axis_index("c")
    u0 = wid * UPW

    iota = lax.iota(jnp.int32, 16)
    cvec = [((dl * 2 + (iota >> 3)) * (CB * 1024) + (iota & 7) * 128)
            for dl in range(2)]

    def fire_gather(u, p):
        t = u // CPT
        c = lax.rem(u, CPT)
        pltpu.sync_copy(x_hbm.at[t, c], idx_v.at[p])
        pltpu.async_copy(table_hbm.at[idx_v.at[p]], rows_v.at[p], gsem.at[p])

    def wait_gather(p):
        pltpu.make_async_copy(table_hbm.at[idx_v.at[p]], rows_v.at[p],
                              gsem.at[p]).wait()

    def store_descs(u, p):
        t = u // CPT
        c = lax.rem(u, CPT)
        return [
            pltpu.make_async_copy(
                trans_v.at[p, pl.ds(dt * CB * 1024, CB * 1024)],
                out_hbm.at[t, dt, pl.ds(c * CB * 1024, CB * 1024)],
                ssem.at[p],
            )
            for dt in range(4)
        ]

    def transpose(p):
        for bt in range(CB):
            iv = (cvec[0] + bt * 1024, cvec[1] + bt * 1024)

            @plsc.parallel_loop(0, 128, unroll=16, carry=iv)
            def _(r2, c):
                iv0, iv1 = c
                r = bt * 128 + r2
                plsc.store_scatter(trans_v.at[p], [iv0],
                                   rows_v[p, r, pl.ds(0, 16)])
                plsc.store_scatter(trans_v.at[p], [iv1],
                                   rows_v[p, r, pl.ds(16, 16)])
                return (iv0 + 1, iv1 + 1)

    fire_gather(u0, 0)
    fire_gather(u0 + 1, 1)

    @pl.loop(0, UPW // 2)
    def _(g):
        for p in range(2):
            u = u0 + g * 2 + p
            wait_gather(p)

            @pl.when(g > 0)
            def _():
                for d in store_descs(u - 2, p):
                    d.wait()

            transpose(p)
            for d in store_descs(u, p):
                d.start()

            @pl.when(g < UPW // 2 - 1)
            def _():
                fire_gather(u + 2, p)

    for p in range(2):
        u_last = u0 + UPW - 2 + p
        for d in store_descs(u_last, p):
            d.wait()


@jax.jit
def kernel(x, table):
    xt = x.T.reshape(HIST_LEN, CPT, CHUNK).astype(jnp.int32)
    flat = _embed(xt, table)
    out5 = flat.reshape(HIST_LEN, 4, 32, 8, 128)
    return out5.transpose(2, 4, 0, 1, 3).reshape(BATCH, HIST_LEN, EMBED_DIM)


# speedup vs baseline: 1.0376x; 1.0376x over previous
import functools

import jax
import jax.numpy as jnp
from jax import lax
from jax.experimental import pallas as pl
from jax.experimental.pallas import tpu as pltpu
from jax.experimental.pallas import tpu_sc as plsc

BATCH = 4096
HIST_LEN = 200
EMBED_DIM = 32

NUM_CORES = 2
NUM_SUBCORES = 16
NUM_WORKERS = NUM_CORES * NUM_SUBCORES

CHUNK = 256
CB = CHUNK // 128
CPT = BATCH // CHUNK
NUM_UNITS = HIST_LEN * CPT
UPW = NUM_UNITS // NUM_WORKERS
TBLK = 4 * CB * 8 * 128


@functools.partial(
    pl.kernel,
    mesh=plsc.VectorSubcoreMesh(core_axis_name="c", subcore_axis_name="s"),
    out_type=jax.ShapeDtypeStruct((HIST_LEN, 4, 32 * 8 * 128), jnp.float32),
    scratch_types=[
        pltpu.VMEM((4, CHUNK), jnp.int32),
        pltpu.VMEM((4, CHUNK, EMBED_DIM), jnp.float32),
        pltpu.VMEM((2, TBLK), jnp.float32),
        pltpu.SemaphoreType.DMA((4,)),
        pltpu.SemaphoreType.DMA((4,)),
        pltpu.SemaphoreType.DMA((2,)),
    ],
    compiler_params=pltpu.CompilerParams(use_tc_tiling_on_sc=False,
                                         needs_layout_passes=False),
)
def _embed(x_hbm, table_hbm, out_hbm, idx_v, rows_v, trans_v, isem, gsem,
           ssem):
    wid = lax.axis_index("s") * NUM_CORES + lax.axis_index("c")
    u0 = wid * UPW

    iota = lax.iota(jnp.int32, 16)
    cvec = [((dl * 2 + (iota >> 3)) * (CB * 1024) + (iota & 7) * 128)
            for dl in range(2)]

    def tc_of(u):
        t = u >> 4
        return t >> 3, t & 7, u & 15

    def idx_descs(u, q):
        tt, st, c = tc_of(u)
        return [
            pltpu.make_async_copy(
                x_hbm.at[tt, c * CB + i, st],
                idx_v.at[q, pl.ds(i * 128, 128)],
                isem.at[q],
            )
            for i in range(CB)
        ]

    def gather_desc(q):
        return pltpu.make_async_copy(table_hbm.at[idx_v.at[q]],
                                     rows_v.at[q], gsem.at[q])

    def store_descs(u, p):
        t = u >> 4
        c = u & 15
        return [
            pltpu.make_async_copy(
                trans_v.at[p, pl.ds(dt * CB * 1024, CB * 1024)],
                out_hbm.at[t, dt, pl.ds(c * CB * 1024, CB * 1024)],
                ssem.at[p],
            )
            for dt in range(4)
        ]

    def transpose(q, p):
        for bt in range(CB):
            iv = (cvec[0] + bt * 1024, cvec[1] + bt * 1024)

            @plsc.parallel_loop(0, 128, unroll=16, carry=iv)
            def _(r2, cr):
                iv0, iv1 = cr
                r = bt * 128 + r2
                plsc.store_scatter(trans_v.at[p], [iv0],
                                   rows_v[q, r, pl.ds(0, 16)])
                plsc.store_scatter(trans_v.at[p], [iv1],
                                   rows_v[q, r, pl.ds(16, 16)])
                return (iv0 + 1, iv1 + 1)

    for q in range(4):
        for d in idx_descs(u0 + q, q):
            d.start()
    for q in range(2):
        for d in idx_descs(u0 + q, q):
            d.wait()
        gather_desc(q).start()

    @pl.loop(0, UPW // 4)
    def _(g):
        for k in range(4):
            q = k
            p = k % 2
            u = u0 + g * 4 + k

            @pl.when(g * 4 + k + 2 < UPW)
            def _():
                for d in idx_descs(u + 2, (k + 2) % 4):
                    d.wait()
                gather_desc((k + 2) % 4).start()

            gather_desc(q).wait()

            @pl.when(g * 4 + k + 4 < UPW)
            def _():
                for d in idx_descs(u + 4, q):
                    d.start()

            @pl.when(g * 4 + k >= 2)
            def _():
                for d in store_descs(u - 2, p):
                    d.wait()

            transpose(q, p)
            for d in store_descs(u, p):
                d.start()

    for k in range(2):
        u_last = u0 + UPW - 2 + k
        for d in store_descs(u_last, k % 2):
            d.wait()


@jax.jit
def kernel(x, table):
    xq = x.T.reshape(25, 8, 32, 128).transpose(0, 2, 1, 3).astype(jnp.int32)
    flat = _embed(xq, table)
    out5 = flat.reshape(HIST_LEN, 4, 32, 8, 128)
    return out5.transpose(2, 4, 0, 1, 3).reshape(BATCH, HIST_LEN, EMBED_DIM)


# speedup vs baseline: 1.5038x; 1.4493x over previous
import functools

import jax
import jax.numpy as jnp
from jax import lax
from jax.experimental import pallas as pl
from jax.experimental.pallas import tpu as pltpu
from jax.experimental.pallas import tpu_sc as plsc

BATCH = 4096
HIST_LEN = 200
EMBED_DIM = 32

NUM_CORES = 2
NUM_SUBCORES = 16
NUM_WORKERS = NUM_CORES * NUM_SUBCORES

CHUNK = 128
CPT = BATCH // CHUNK
NUM_UNITS = HIST_LEN * CPT
UPW = NUM_UNITS // NUM_WORKERS
RS = 129
DTS = 8 * RS


@functools.partial(
    pl.kernel,
    mesh=plsc.VectorSubcoreMesh(core_axis_name="c", subcore_axis_name="s"),
    out_type=jax.ShapeDtypeStruct((HIST_LEN, 4, 32, 8, 128), jnp.float32),
    scratch_types=[
        pltpu.VMEM((4, CHUNK), jnp.int32),
        pltpu.VMEM((4, CHUNK, EMBED_DIM), jnp.float32),
        pltpu.VMEM((2, 4, 8, RS), jnp.float32),
        pltpu.SemaphoreType.DMA((4,)),
        pltpu.SemaphoreType.DMA((4,)),
        pltpu.SemaphoreType.DMA((2,)),
    ],
    compiler_params=pltpu.CompilerParams(use_tc_tiling_on_sc=False,
                                         needs_layout_passes=False),
)
def _embed(x_hbm, table_hbm, out_hbm, idx_v, rows_v, trans_v, isem, gsem,
           ssem):
    wid = lax.axis_index("s") * NUM_CORES + lax.axis_index("c")
    u0 = wid * UPW

    iota = lax.iota(jnp.int32, 16)
    cdt = [(dl * 2 + (iota >> 3)) for dl in range(2)]
    cds = iota & 7

    def tc_of(u):
        t = u >> 5
        return t >> 3, t & 7, u & 31

    def idx_desc(u, q):
        tt, st, c = tc_of(u)
        return pltpu.make_async_copy(x_hbm.at[tt, c, st], idx_v.at[q],
                                     isem.at[q])

    def gather_desc(q):
        return pltpu.make_async_copy(table_hbm.at[idx_v.at[q]],
                                     rows_v.at[q], gsem.at[q])

    def store_descs(u, p):
        t = u >> 5
        c = u & 31
        return [
            pltpu.make_async_copy(
                trans_v.at[p, dt, :, pl.ds(0, 128)],
                out_hbm.at[t, dt, c],
                ssem.at[p],
            )
            for dt in range(4)
        ]

    def transpose(q, p):
        @plsc.parallel_loop(0, CHUNK, unroll=16, carry=iota * 0)
        def _(r, lvec):
            plsc.store_scatter(trans_v.at[p], [cdt[0], cds, lvec],
                               rows_v[q, r, pl.ds(0, 16)])
            plsc.store_scatter(trans_v.at[p], [cdt[1], cds, lvec],
                               rows_v[q, r, pl.ds(16, 16)])
            return lvec + 1

    for q in range(4):
        idx_desc(u0 + q, q).start()
    for q in range(2):
        idx_desc(u0 + q, q).wait()
        gather_desc(q).start()

    @pl.loop(0, UPW // 4)
    def _(g):
        for k in range(4):
            q = k
            p = k % 2
            u = u0 + g * 4 + k

            @pl.when(g * 4 + k + 2 < UPW)
            def _():
                idx_desc(u + 2, (k + 2) % 4).wait()
                gather_desc((k + 2) % 4).start()

            gather_desc(q).wait()

            @pl.when(g * 4 + k + 4 < UPW)
            def _():
                idx_desc(u + 4, q).start()

            @pl.when(g * 4 + k >= 2)
            def _():
                for d in store_descs(u - 2, p):
                    d.wait()

            transpose(q, p)
            for d in store_descs(u, p):
                d.start()

    for k in range(2):
        u_last = u0 + UPW - 2 + k
        for d in store_descs(u_last, k % 2):
            d.wait()


@jax.jit
def kernel(x, table):
    xq = x.T.reshape(25, 8, 32, 128).transpose(0, 2, 1, 3).astype(jnp.int32)
    out5 = _embed(xq, table)
    return out5.transpose(2, 4, 0, 1, 3).reshape(BATCH, HIST_LEN, EMBED_DIM)
